# Initial kernel scaffold; baseline (speedup 1.0000x reference)
import jax
import jax.numpy as jnp
from jax.experimental import pallas as pl


def kernel(node_features, edge_index, edge_type, edge_norm, basis, att, root, bias1, w_nbr, w_lin, bias2):
    raise NotImplementedError("write your pallas kernel here")


# speedup vs baseline: 7.4476x; 7.4476x over previous
import functools

import jax
import jax.numpy as jnp
from jax import lax
from jax.experimental import pallas as pl
from jax.experimental.pallas import tpu as pltpu
from jax.experimental.pallas import tpu_sc as plsc

_N = 10000
_E = 320000
_D = 128
_H1 = 128
_H2 = 128
_R = 8
_NB = 30

_NC = 2
_NS = 16
_NW = _NC * _NS
_CH = 128
_NCHUNK = _E // _CH
_Q, _REM = divmod(_NCHUNK, _NW)
_ROWS_PER_TILE = _N // _NS
_W1 = _H1 + 16

_mesh = plsc.VectorSubcoreMesh(
    core_axis_name="c", subcore_axis_name="s", num_cores=_NC, num_subcores=_NS)


def _dense_a_body(att_ref, basis_ref, x_ref, root_ref, b1_ref, xw_ref, xr_ref):
    r = pl.program_id(0)

    def bstep(b, acc):
        return acc + att_ref[r, b] * basis_ref[b]

    wr = lax.fori_loop(0, _NB, bstep, jnp.zeros((_D, _H1), jnp.float32))
    xw_ref[0] = jnp.dot(x_ref[...], wr, preferred_element_type=jnp.float32)

    @pl.when(r == 0)
    def _():
        xr_ref[...] = (jnp.dot(x_ref[...], root_ref[...],
                               preferred_element_type=jnp.float32)
                       + b1_ref[...])


_dense_a = pl.pallas_call(
    _dense_a_body,
    grid=(_R,),
    in_specs=[
        pl.BlockSpec(memory_space=pltpu.SMEM),
        pl.BlockSpec((_NB, _D, _H1), lambda r: (0, 0, 0)),
        pl.BlockSpec((_N, _D), lambda r: (0, 0)),
        pl.BlockSpec((_D, _H1), lambda r: (0, 0)),
        pl.BlockSpec((1, _H1), lambda r: (0, 0)),
    ],
    out_specs=[
        pl.BlockSpec((1, _N, _H1), lambda r: (r, 0, 0)),
        pl.BlockSpec((_N, _H1), lambda r: (0, 0)),
    ],
    out_shape=[
        jax.ShapeDtypeStruct((_R, _N, _H1), jnp.float32),
        jax.ShapeDtypeStruct((_N, _H1), jnp.float32),
    ],
)


def _sc1_body(table, src2, typ2, dst2, norm2, zinit, out,
              acc, src_v, typ_v, gidx_v, dst_v, norm_v, gbuf, sbuf, sem):
    c = lax.axis_index("c")
    s = lax.axis_index("s")
    w = s * _NC + c
    start = w * _Q + jnp.minimum(w, _REM)
    mycnt = _Q + (w < _REM).astype(jnp.int32)

    rbase = s * _ROWS_PER_TILE
    pltpu.sync_copy(zinit.at[pl.ds(rbase, _ROWS_PER_TILE)],
                    acc.at[pl.ds(rbase, _ROWS_PER_TILE)])

    def ones_row(i, carry):
        sbuf[i, pl.ds(_H1, 16)] = jnp.ones((16,), jnp.float32)
        return carry
    lax.fori_loop(0, _CH, ones_row, 0)

    plsc.subcore_barrier()

    def chunk_body(j, carry):
        ch = start + j
        pltpu.sync_copy(src2.at[ch], src_v)
        pltpu.sync_copy(typ2.at[ch], typ_v)
        pltpu.sync_copy(dst2.at[ch], dst_v)
        pltpu.sync_copy(norm2.at[ch], norm_v)
        for g in range(_CH // 16):
            sl = pl.ds(16 * g, 16)
            gidx_v[sl] = typ_v[sl] * _N + src_v[sl]
        pltpu.async_copy(table.at[gidx_v], gbuf, sem).wait()

        def egroup(q, c2):
            base = q * 16
            nv = norm_v[pl.ds(base, 16)]
            for i in range(16):
                nrm = nv[i]
                e = base + i
                for g in range(_H1 // 16):
                    sl = pl.ds(16 * g, 16)
                    sbuf[e, sl] = gbuf[e, sl] * nrm
            return c2
        lax.fori_loop(0, _CH // 16, egroup, 0)
        pltpu.sync_copy(sbuf, acc.at[dst_v], add=True)
        return carry

    lax.fori_loop(0, mycnt, chunk_body, 0)

    plsc.subcore_barrier()
    pltpu.sync_copy(acc.at[pl.ds(rbase, _ROWS_PER_TILE)],
                    out.at[c, pl.ds(rbase, _ROWS_PER_TILE)])


_sc_pass1 = functools.partial(
    pl.kernel,
    out_type=jax.ShapeDtypeStruct((_NC, _N, _W1), jnp.float32),
    mesh=_mesh,
    compiler_params=pltpu.CompilerParams(use_tc_tiling_on_sc=False),
    scratch_types=[
        pltpu.VMEM_SHARED((_N, _W1), jnp.float32),
        pltpu.VMEM((_CH,), jnp.int32),
        pltpu.VMEM((_CH,), jnp.int32),
        pltpu.VMEM((_CH,), jnp.int32),
        pltpu.VMEM((_CH,), jnp.int32),
        pltpu.VMEM((_CH,), jnp.float32),
        pltpu.VMEM((_CH, _H1), jnp.float32),
        pltpu.VMEM((_CH, _W1), jnp.float32),
        pltpu.SemaphoreType.DMA,
    ],
)(_sc1_body)


def _dense_c_body(p_ref, xr_ref, wn_ref, wl_ref, b2_ref, h_ref, y2_ref):
    sums = p_ref[0, :, :_H1] + p_ref[1, :, :_H1]
    cnt = p_ref[0, :, _H1:] + p_ref[1, :, _H1:]
    cnt0 = jnp.max(cnt, axis=1, keepdims=True)
    x1 = sums / jnp.maximum(cnt0, 1.0) + xr_ref[...]
    h_ref[...] = jnp.dot(x1, wn_ref[...], preferred_element_type=jnp.float32)
    y2_ref[...] = (jnp.dot(x1, wl_ref[...], preferred_element_type=jnp.float32)
                   + b2_ref[...])


_dense_c = pl.pallas_call(
    _dense_c_body,
    out_shape=[
        jax.ShapeDtypeStruct((_N, _H1), jnp.float32),
        jax.ShapeDtypeStruct((_N, _H2), jnp.float32),
    ],
)


def _sc2_body(table, src2, dst2, zinit, out, acc, src_v, dst_v, buf, sem):
    c = lax.axis_index("c")
    s = lax.axis_index("s")
    w = s * _NC + c
    start = w * _Q + jnp.minimum(w, _REM)
    mycnt = _Q + (w < _REM).astype(jnp.int32)

    rbase = s * _ROWS_PER_TILE
    pltpu.sync_copy(zinit.at[pl.ds(rbase, _ROWS_PER_TILE)],
                    acc.at[pl.ds(rbase, _ROWS_PER_TILE)])
    plsc.subcore_barrier()

    def chunk_body(j, carry):
        ch = start + j
        pltpu.sync_copy(src2.at[ch], src_v)
        pltpu.sync_copy(dst2.at[ch], dst_v)
        pltpu.async_copy(table.at[src_v], buf, sem).wait()
        pltpu.sync_copy(buf, acc.at[dst_v], add=True)
        return carry

    lax.fori_loop(0, mycnt, chunk_body, 0)

    plsc.subcore_barrier()
    pltpu.sync_copy(acc.at[pl.ds(rbase, _ROWS_PER_TILE)],
                    out.at[c, pl.ds(rbase, _ROWS_PER_TILE)])


_sc_pass2 = functools.partial(
    pl.kernel,
    out_type=jax.ShapeDtypeStruct((_NC, _N, _H2), jnp.float32),
    mesh=_mesh,
    compiler_params=pltpu.CompilerParams(use_tc_tiling_on_sc=False),
    scratch_types=[
        pltpu.VMEM_SHARED((_N, _H2), jnp.float32),
        pltpu.VMEM((_CH,), jnp.int32),
        pltpu.VMEM((_CH,), jnp.int32),
        pltpu.VMEM((_CH, _H2), jnp.float32),
        pltpu.SemaphoreType.DMA,
    ],
)(_sc2_body)


def _dense_e_body(q_ref, y2_ref, o_ref):
    o_ref[...] = q_ref[0] + q_ref[1] + y2_ref[...]


_dense_e = pl.pallas_call(
    _dense_e_body,
    out_shape=jax.ShapeDtypeStruct((_N, _H2), jnp.float32),
)


def kernel(node_features, edge_index, edge_type, edge_norm, basis, att, root,
           bias1, w_nbr, w_lin, bias2):
    src2 = edge_index[0].reshape(_NCHUNK, _CH)
    typ2 = edge_type.reshape(_NCHUNK, _CH)
    dst2 = edge_index[1].reshape(_NCHUNK, _CH)
    norm2 = edge_norm.reshape(_NCHUNK, _CH)
    zero1 = jnp.zeros((_N, _W1), jnp.float32)
    zero2 = jnp.zeros((_N, _H2), jnp.float32)

    xw, xr = _dense_a(att, basis, node_features, root, bias1.reshape(1, _H1))
    xw_flat = xw.reshape(_R * _N, _H1)
    part1 = _sc_pass1(xw_flat, src2, typ2, dst2, norm2, zero1)
    h, y2 = _dense_c(part1, xr, w_nbr, w_lin, bias2.reshape(1, _H2))
    part2 = _sc_pass2(h, src2, dst2, zero2)
    return _dense_e(part2, y2)


# speedup vs baseline: 20.1152x; 2.7009x over previous
import functools

import jax
import jax.numpy as jnp
from jax import lax
from jax.experimental import pallas as pl
from jax.experimental.pallas import tpu as pltpu
from jax.experimental.pallas import tpu_sc as plsc

_N = 10000
_E = 320000
_D = 128
_H1 = 128
_H2 = 128
_R = 8
_NB = 30

_NC = 2
_NS = 16
_NW = _NC * _NS
_CH = 128
_NCHUNK = _E // _CH
_Q, _REM = divmod(_NCHUNK, _NW)
_ROWS_PER_TILE = _N // _NS
_CW = 16

_mesh = plsc.VectorSubcoreMesh(
    core_axis_name="c", subcore_axis_name="s", num_cores=_NC, num_subcores=_NS)
_sc_params = pltpu.CompilerParams(use_tc_tiling_on_sc=False)


def _dense_a_body(att_ref, basis_ref, x_ref, root_ref, b1_ref, xw_ref, xr_ref):
    r = pl.program_id(0)

    def bstep(b, acc):
        return acc + att_ref[r, b] * basis_ref[b]

    wr = lax.fori_loop(0, _NB, bstep, jnp.zeros((_D, _H1), jnp.float32))
    xw_ref[0] = jnp.dot(x_ref[...], wr, preferred_element_type=jnp.float32)

    @pl.when(r == 0)
    def _():
        xr_ref[...] = (jnp.dot(x_ref[...], root_ref[...],
                               preferred_element_type=jnp.float32)
                       + b1_ref[...])


_dense_a = pl.pallas_call(
    _dense_a_body,
    grid=(_R,),
    in_specs=[
        pl.BlockSpec(memory_space=pltpu.SMEM),
        pl.BlockSpec((_NB, _D, _H1), lambda r: (0, 0, 0)),
        pl.BlockSpec((_N, _D), lambda r: (0, 0)),
        pl.BlockSpec((_D, _H1), lambda r: (0, 0)),
        pl.BlockSpec((1, _H1), lambda r: (0, 0)),
    ],
    out_specs=[
        pl.BlockSpec((1, _N, _H1), lambda r: (r, 0, 0)),
        pl.BlockSpec((_N, _H1), lambda r: (0, 0)),
    ],
    out_shape=[
        jax.ShapeDtypeStruct((_R, _N, _H1), jnp.float32),
        jax.ShapeDtypeStruct((_N, _H1), jnp.float32),
    ],
)


def _worker_range(c, s):
    w = s * _NC + c
    start = w * _Q + jnp.minimum(w, _REM)
    mycnt = _Q + (w < _REM).astype(jnp.int32)
    return start, mycnt


def _sc1_body(table, src2, typ2, dst2, norm2, zsum, zcnt, outs, outc,
              acc, acc_cnt,
              srcr0, srcr1, typr0, typr1, dstr0, dstr1, normr0, normr1,
              gidx0, gidx1, gbuf0, gbuf1, obuf,
              semr0, semr1, semg0, semg1, sems0, sems1, semo, sem_ld):
    c = lax.axis_index("c")
    s = lax.axis_index("s")
    start, mycnt = _worker_range(c, s)
    rbase = s * _ROWS_PER_TILE

    pltpu.make_async_copy(zsum.at[pl.ds(rbase, _ROWS_PER_TILE)],
                          acc.at[pl.ds(rbase, _ROWS_PER_TILE)], sem_ld).start()
    pltpu.make_async_copy(zcnt.at[pl.ds(rbase, _ROWS_PER_TILE)],
                          acc_cnt.at[pl.ds(rbase, _ROWS_PER_TILE)],
                          sem_ld).start()

    def ones_row(i, carry):
        obuf[i, :] = jnp.ones((_CW,), jnp.float32)
        return carry
    lax.fori_loop(0, _CH, ones_row, 0)

    srcr = (srcr0, srcr1)
    typr = (typr0, typr1)
    dstr = (dstr0, dstr1)
    normr = (normr0, normr1)
    gidxr = (gidx0, gidx1)
    bufs = (gbuf0, gbuf1)
    rsems = (semr0, semr1)
    gsems = (semg0, semg1)
    ssems = (sems0, sems1)

    def rows(ch, b):
        return (pltpu.make_async_copy(src2.at[ch], srcr[b], rsems[b]),
                pltpu.make_async_copy(typ2.at[ch], typr[b], rsems[b]),
                pltpu.make_async_copy(dst2.at[ch], dstr[b], rsems[b]),
                pltpu.make_async_copy(norm2.at[ch], normr[b], rsems[b]))

    def gather(b):
        return pltpu.make_async_copy(table.at[gidxr[b]], bufs[b], gsems[b])

    def scat(b):
        return pltpu.make_async_copy(bufs[b], acc.at[dstr[b]], ssems[b])

    def scat_ones(b):
        return pltpu.make_async_copy(obuf, acc_cnt.at[dstr[b]], semo)

    def make_gidx(b):
        for g in range(_CH // 16):
            sl = pl.ds(16 * g, 16)
            gidxr[b][sl] = typr[b][sl] * _N + srcr[b][sl]

    pltpu.make_async_copy(zsum.at[pl.ds(rbase, _ROWS_PER_TILE)],
                          acc.at[pl.ds(rbase, _ROWS_PER_TILE)], sem_ld).wait()
    pltpu.make_async_copy(zcnt.at[pl.ds(rbase, _ROWS_PER_TILE)],
                          acc_cnt.at[pl.ds(rbase, _ROWS_PER_TILE)],
                          sem_ld).wait()
    plsc.subcore_barrier()

    @pl.when(mycnt > 0)
    def _():
        for d in rows(start, 0):
            d.start()
        for d in rows(start, 0):
            d.wait()
        make_gidx(0)
        gather(0).start()

    def chunk_body(j, carry):
        def arm(b):
            ob = 1 - b

            @pl.when(j >= 1)
            def _():
                scat(ob).wait()
                scat_ones(ob).wait()

            @pl.when(j + 1 < mycnt)
            def _():
                for d in rows(start + j + 1, ob):
                    d.start()

            gather(b).wait()

            @pl.when(j + 1 < mycnt)
            def _():
                for d in rows(start + j + 1, ob):
                    d.wait()
                make_gidx(ob)
                gather(ob).start()

            def egroup(q, c2):
                nv = normr[b][pl.ds(16 * q, 16)]
                for i in range(16):
                    nrm = nv[i]
                    e = q * 16 + i
                    for g in range(_H1 // 16):
                        sl = pl.ds(16 * g, 16)
                        bufs[b][e, sl] = bufs[b][e, sl] * nrm
                return c2
            lax.fori_loop(0, _CH // 16, egroup, 0)

            scat(b).start(add=True)
            scat_ones(b).start(add=True)

        @pl.when(j % 2 == 0)
        def _():
            arm(0)

        @pl.when(j % 2 == 1)
        def _():
            arm(1)

        return carry

    lax.fori_loop(0, mycnt, chunk_body, 0)

    @pl.when(mycnt > 0)
    def _():
        @pl.when((mycnt - 1) % 2 == 0)
        def _():
            scat(0).wait()

        @pl.when((mycnt - 1) % 2 == 1)
        def _():
            scat(1).wait()

        scat_ones(0).wait()

    plsc.subcore_barrier()
    pltpu.sync_copy(acc.at[pl.ds(rbase, _ROWS_PER_TILE)],
                    outs.at[c, pl.ds(rbase, _ROWS_PER_TILE)])
    pltpu.sync_copy(acc_cnt.at[pl.ds(rbase, _ROWS_PER_TILE)],
                    outc.at[c, pl.ds(rbase, _ROWS_PER_TILE)])


_sc_pass1 = functools.partial(
    pl.kernel,
    out_type=(
        jax.ShapeDtypeStruct((_NC, _N, _H1), jnp.float32),
        jax.ShapeDtypeStruct((_NC, _N, _CW), jnp.float32),
    ),
    mesh=_mesh,
    compiler_params=_sc_params,
    scratch_types=[
        pltpu.VMEM_SHARED((_N, _H1), jnp.float32),
        pltpu.VMEM_SHARED((_N, _CW), jnp.float32),
        pltpu.VMEM((_CH,), jnp.int32),
        pltpu.VMEM((_CH,), jnp.int32),
        pltpu.VMEM((_CH,), jnp.int32),
        pltpu.VMEM((_CH,), jnp.int32),
        pltpu.VMEM((_CH,), jnp.int32),
        pltpu.VMEM((_CH,), jnp.int32),
        pltpu.VMEM((_CH,), jnp.float32),
        pltpu.VMEM((_CH,), jnp.float32),
        pltpu.VMEM((_CH,), jnp.int32),
        pltpu.VMEM((_CH,), jnp.int32),
        pltpu.VMEM((_CH, _H1), jnp.float32),
        pltpu.VMEM((_CH, _H1), jnp.float32),
        pltpu.VMEM((_CH, _CW), jnp.float32),
        pltpu.SemaphoreType.DMA,
        pltpu.SemaphoreType.DMA,
        pltpu.SemaphoreType.DMA,
        pltpu.SemaphoreType.DMA,
        pltpu.SemaphoreType.DMA,
        pltpu.SemaphoreType.DMA,
        pltpu.SemaphoreType.DMA,
        pltpu.SemaphoreType.DMA,
    ],
)(_sc1_body)


def _dense_c_body(p_ref, c_ref, xr_ref, wn_ref, wl_ref, b2_ref, h_ref, y2_ref):
    sums = p_ref[0] + p_ref[1]
    cnt = c_ref[0] + c_ref[1]
    cnt0 = jnp.max(cnt, axis=1, keepdims=True)
    x1 = sums / jnp.maximum(cnt0, 1.0) + xr_ref[...]
    h_ref[...] = jnp.dot(x1, wn_ref[...], preferred_element_type=jnp.float32)
    y2_ref[...] = (jnp.dot(x1, wl_ref[...], preferred_element_type=jnp.float32)
                   + b2_ref[...])


_dense_c = pl.pallas_call(
    _dense_c_body,
    out_shape=[
        jax.ShapeDtypeStruct((_N, _H1), jnp.float32),
        jax.ShapeDtypeStruct((_N, _H2), jnp.float32),
    ],
)


def _sc2_body(table, src2, dst2, zsum, out,
              acc, srcr0, srcr1, dstr0, dstr1, gbuf0, gbuf1,
              semr0, semr1, semg0, semg1, sems0, sems1, sem_ld):
    c = lax.axis_index("c")
    s = lax.axis_index("s")
    start, mycnt = _worker_range(c, s)
    rbase = s * _ROWS_PER_TILE

    pltpu.make_async_copy(zsum.at[pl.ds(rbase, _ROWS_PER_TILE)],
                          acc.at[pl.ds(rbase, _ROWS_PER_TILE)], sem_ld).start()

    srcr = (srcr0, srcr1)
    dstr = (dstr0, dstr1)
    bufs = (gbuf0, gbuf1)
    rsems = (semr0, semr1)
    gsems = (semg0, semg1)
    ssems = (sems0, sems1)

    def rows(ch, b):
        return (pltpu.make_async_copy(src2.at[ch], srcr[b], rsems[b]),
                pltpu.make_async_copy(dst2.at[ch], dstr[b], rsems[b]))

    def gather(b):
        return pltpu.make_async_copy(table.at[srcr[b]], bufs[b], gsems[b])

    def scat(b):
        return pltpu.make_async_copy(bufs[b], acc.at[dstr[b]], ssems[b])

    pltpu.make_async_copy(zsum.at[pl.ds(rbase, _ROWS_PER_TILE)],
                          acc.at[pl.ds(rbase, _ROWS_PER_TILE)], sem_ld).wait()
    plsc.subcore_barrier()

    @pl.when(mycnt > 0)
    def _():
        for d in rows(start, 0):
            d.start()
        for d in rows(start, 0):
            d.wait()
        gather(0).start()

    def chunk_body(j, carry):
        def arm(b):
            ob = 1 - b

            @pl.when(j >= 1)
            def _():
                scat(ob).wait()

            @pl.when(j + 1 < mycnt)
            def _():
                for d in rows(start + j + 1, ob):
                    d.start()

            gather(b).wait()

            @pl.when(j + 1 < mycnt)
            def _():
                for d in rows(start + j + 1, ob):
                    d.wait()
                gather(ob).start()

            scat(b).start(add=True)

        @pl.when(j % 2 == 0)
        def _():
            arm(0)

        @pl.when(j % 2 == 1)
        def _():
            arm(1)

        return carry

    lax.fori_loop(0, mycnt, chunk_body, 0)

    @pl.when(mycnt > 0)
    def _():
        @pl.when((mycnt - 1) % 2 == 0)
        def _():
            scat(0).wait()

        @pl.when((mycnt - 1) % 2 == 1)
        def _():
            scat(1).wait()

    plsc.subcore_barrier()
    pltpu.sync_copy(acc.at[pl.ds(rbase, _ROWS_PER_TILE)],
                    out.at[c, pl.ds(rbase, _ROWS_PER_TILE)])


_sc_pass2 = functools.partial(
    pl.kernel,
    out_type=jax.ShapeDtypeStruct((_NC, _N, _H2), jnp.float32),
    mesh=_mesh,
    compiler_params=_sc_params,
    scratch_types=[
        pltpu.VMEM_SHARED((_N, _H2), jnp.float32),
        pltpu.VMEM((_CH,), jnp.int32),
        pltpu.VMEM((_CH,), jnp.int32),
        pltpu.VMEM((_CH,), jnp.int32),
        pltpu.VMEM((_CH,), jnp.int32),
        pltpu.VMEM((_CH, _H2), jnp.float32),
        pltpu.VMEM((_CH, _H2), jnp.float32),
        pltpu.SemaphoreType.DMA,
        pltpu.SemaphoreType.DMA,
        pltpu.SemaphoreType.DMA,
        pltpu.SemaphoreType.DMA,
        pltpu.SemaphoreType.DMA,
        pltpu.SemaphoreType.DMA,
        pltpu.SemaphoreType.DMA,
    ],
)(_sc2_body)


def _dense_e_body(q_ref, y2_ref, o_ref):
    o_ref[...] = q_ref[0] + q_ref[1] + y2_ref[...]


_dense_e = pl.pallas_call(
    _dense_e_body,
    out_shape=jax.ShapeDtypeStruct((_N, _H2), jnp.float32),
)


def kernel(node_features, edge_index, edge_type, edge_norm, basis, att, root,
           bias1, w_nbr, w_lin, bias2):
    src2 = edge_index[0].reshape(_NCHUNK, _CH)
    typ2 = edge_type.reshape(_NCHUNK, _CH)
    dst2 = edge_index[1].reshape(_NCHUNK, _CH)
    norm2 = edge_norm.reshape(_NCHUNK, _CH)
    zsum = jnp.zeros((_N, _H1), jnp.float32)
    zcnt = jnp.zeros((_N, _CW), jnp.float32)

    xw, xr = _dense_a(att, basis, node_features, root, bias1.reshape(1, _H1))
    xw_flat = xw.reshape(_R * _N, _H1)
    part1, cnt1 = _sc_pass1(xw_flat, src2, typ2, dst2, norm2, zsum, zcnt)
    h, y2 = _dense_c(part1, cnt1, xr, w_nbr, w_lin, bias2.reshape(1, _H2))
    part2 = _sc_pass2(h, src2, dst2, zsum)
    return _dense_e(part2, y2)
